# baseline (speedup 1.0000x reference)
import functools

import jax
import jax.numpy as jnp
from jax.experimental import pallas as pl
from jax.experimental.pallas import tpu as pltpu

_GAMMA = 16.0


def _choose_chunk(seg_len: int, max_chunk: int = 10000) -> int:
    best = 0
    for ch in range(8, min(seg_len, max_chunk) + 1, 8):
        if seg_len % ch == 0:
            best = ch
    return best if best else seg_len


def _bf16_dot(a, b, contract):
    return jax.lax.dot_general(
        a.astype(jnp.bfloat16), b.astype(jnp.bfloat16),
        (contract, ((), ())),
        preferred_element_type=jnp.float32)


def _ones_sum(a):
    ones = jnp.ones((1, a.shape[0]), jnp.float32)
    return jax.lax.dot_general(ones, a, (((1,), (0,)), ((), ())),
                               precision=jax.lax.Precision.HIGHEST,
                               preferred_element_type=jnp.float32)


def _attn_body(p_ref, x_ref, s_ref, np_out, nx_out,
               m_s, l_s, accp_s, accx_s, np_s, pn2_s, *, nch, gamma):
    s = pl.program_id(1)
    j = pl.program_id(2)
    S = s_ref[...]
    xc = x_ref[...]
    pct = p_ref[0]

    w_h = _bf16_dot(xc, S, ((1,), (1,)))

    @pl.when(s == 0)
    def _sweep_max_sum():
        @pl.when(j == 0)
        def _():
            m_s[...] = jnp.full_like(m_s[...], -jnp.inf)
            l_s[...] = jnp.zeros_like(l_s[...])
        m_s[...] = jnp.maximum(m_s[...], jnp.max(w_h, axis=0, keepdims=True))
        e = jnp.exp(w_h - 20.0)
        l_s[...] = l_s[...] + jnp.sum(e, axis=0, keepdims=True)

        @pl.when(j == nch - 1)
        def _fin_l():
            l_s[...] = l_s[...] * jnp.exp(20.0 - m_s[...])

    @pl.when(s == 1)
    def _sweep_np():
        @pl.when(j == 0)
        def _():
            accp_s[...] = jnp.zeros_like(accp_s[...])
        probs = jnp.exp(w_h - m_s[...]) / l_s[...]
        accp_s[...] = accp_s[...] + _bf16_dot(pct, probs, ((1,), (0,)))

        @pl.when(j == nch - 1)
        def _fin_np():
            np_t = accp_s[...]
            np_s[...] = np_t
            pn2_s[...] = -gamma * jnp.sum(np_t * np_t, axis=0, keepdims=True)
            np_out[...] = np_t.T

    @pl.when(s == 2)
    def _sweep_nx():
        @pl.when(j == 0)
        def _():
            m_s[...] = jnp.full_like(m_s[...], -jnp.inf)
            l_s[...] = jnp.zeros_like(l_s[...])
            accx_s[...] = jnp.zeros_like(accx_s[...])
        cross = _bf16_dot(pct, np_s[...], ((0,), (0,)))
        psq = pct * pct
        negg3 = jnp.full((3, 1), -gamma, jnp.float32)
        pc2g = jax.lax.dot_general(psq, negg3, (((0,), (0,)), ((), ())),
                                   precision=jax.lax.Precision.HIGHEST,
                                   preferred_element_type=jnp.float32)
        nd = ((2.0 * gamma) * cross + pn2_s[...]) + pc2g
        w = jnp.exp(nd) * w_h
        m_old = m_s[...]
        m_new = jnp.maximum(m_old, jnp.max(w, axis=0, keepdims=True))
        alpha = jnp.exp(m_old - m_new)
        e = jnp.exp(w - m_new)
        m_s[...] = m_new
        l_s[...] = l_s[...] * alpha + jnp.sum(e, axis=0, keepdims=True)
        ev = _bf16_dot(xc, e, ((0,), (0,)))
        accx_s[...] = accx_s[...] * alpha + ev

        @pl.when(j == nch - 1)
        def _fin_nx():
            nx_out[...] = (accx_s[...] / l_s[...]).T


def kernel(p, x, o, S):
    n = p.shape[0]
    n_seg = o.shape[0]
    seg_len = n // n_seg
    n_s = S.shape[0]
    c = x.shape[1]
    ch = _choose_chunk(seg_len)
    nch = seg_len // ch

    body = functools.partial(_attn_body, nch=nch, gamma=_GAMMA)
    n_p, n_x = pl.pallas_call(
        body,
        grid=(n_seg, 3, nch),
        in_specs=[
            pl.BlockSpec((1, 3, ch), lambda i, s, j: (i * nch + j, 0, 0)),
            pl.BlockSpec((ch, c), lambda i, s, j: (i * nch + j, 0)),
            pl.BlockSpec((n_s, c), lambda i, s, j: (0, 0)),
        ],
        out_specs=[
            pl.BlockSpec((n_s, 3), lambda i, s, j: (i, 0)),
            pl.BlockSpec((n_s, c), lambda i, s, j: (i, 0)),
        ],
        out_shape=[
            jax.ShapeDtypeStruct((n_seg * n_s, 3), jnp.float32),
            jax.ShapeDtypeStruct((n_seg * n_s, c), jnp.float32),
        ],
        scratch_shapes=[
            pltpu.VMEM((1, n_s), jnp.float32),
            pltpu.VMEM((1, n_s), jnp.float32),
            pltpu.VMEM((3, n_s), jnp.float32),
            pltpu.VMEM((c, n_s), jnp.float32),
            pltpu.VMEM((3, n_s), jnp.float32),
            pltpu.VMEM((1, n_s), jnp.float32),
        ],
    )(p.T.reshape(3, n // ch, ch).swapaxes(0, 1),
      x.astype(jnp.bfloat16), S.astype(jnp.bfloat16))

    n_o = jnp.arange(1, n_seg + 1, dtype=jnp.int32) * jnp.int32(n_s)
    return (n_p, n_x, n_o)


# speedup vs baseline: 1.1783x; 1.1783x over previous
import functools

import jax
import jax.numpy as jnp
from jax.experimental import pallas as pl
from jax.experimental.pallas import tpu as pltpu

_GAMMA = 16.0


def _choose_chunk(seg_len: int, max_chunk: int = 10000) -> int:
    best = 0
    for ch in range(8, min(seg_len, max_chunk) + 1, 8):
        if seg_len % ch == 0:
            best = ch
    return best if best else seg_len


def _bf16_dot(a, b, contract):
    return jax.lax.dot_general(
        a.astype(jnp.bfloat16), b.astype(jnp.bfloat16),
        (contract, ((), ())),
        preferred_element_type=jnp.float32)


def _ones_sum(a):
    ones = jnp.ones((1, a.shape[0]), jnp.float32)
    return jax.lax.dot_general(ones, a, (((1,), (0,)), ((), ())),
                               precision=jax.lax.Precision.HIGHEST,
                               preferred_element_type=jnp.float32)


def _attn_body(p_ref, x_ref, s_ref, np_out, nx_out,
               m_s, l_s, accp_s, accx_s, np_s, pn2_s, *, nch, gamma):
    s = pl.program_id(1)
    j = pl.program_id(2)
    S = s_ref[...]
    xc = x_ref[...]
    pc = p_ref[...]

    w_h = _bf16_dot(xc, S, ((1,), (1,)))

    @pl.when(s == 0)
    def _sweep_max_sum():
        @pl.when(j == 0)
        def _():
            m_s[...] = jnp.full_like(m_s[...], -jnp.inf)
            l_s[...] = jnp.zeros_like(l_s[...])
        m_s[...] = jnp.maximum(m_s[...], jnp.max(w_h, axis=0, keepdims=True))
        e = jnp.exp(w_h - 20.0)
        l_s[...] = l_s[...] + jnp.sum(e, axis=0, keepdims=True)

        @pl.when(j == nch - 1)
        def _fin_l():
            l_s[...] = l_s[...] * jnp.exp(20.0 - m_s[...])

    @pl.when(s == 1)
    def _sweep_np():
        @pl.when(j == 0)
        def _():
            accp_s[...] = jnp.zeros_like(accp_s[...])
        probs = jnp.exp(w_h - m_s[...]) / l_s[...]
        accp_s[...] = accp_s[...] + _bf16_dot(pc, probs, ((0,), (0,)))

        @pl.when(j == nch - 1)
        def _fin_np():
            np_t = accp_s[...]
            np_s[...] = np_t
            pn2_s[...] = -gamma * jnp.sum(np_t * np_t, axis=0, keepdims=True)
            np_out[...] = np_t.T

    @pl.when(s == 2)
    def _sweep_nx():
        @pl.when(j == 0)
        def _():
            m_s[...] = jnp.full_like(m_s[...], -jnp.inf)
            l_s[...] = jnp.zeros_like(l_s[...])
            accx_s[...] = jnp.zeros_like(accx_s[...])
        cross = _bf16_dot(pc, np_s[...], ((1,), (0,)))
        psq = pc * pc
        negg3 = jnp.full((1, 3), -gamma, jnp.float32)
        pc2g = jax.lax.dot_general(psq, negg3, (((1,), (1,)), ((), ())),
                                   precision=jax.lax.Precision.HIGHEST,
                                   preferred_element_type=jnp.float32)
        nd = ((2.0 * gamma) * cross + pn2_s[...]) + pc2g
        w = jnp.exp(nd) * w_h
        m_old = m_s[...]
        m_new = jnp.maximum(m_old, jnp.max(w, axis=0, keepdims=True))
        alpha = jnp.exp(m_old - m_new)
        e = jnp.exp(w - m_new)
        m_s[...] = m_new
        l_s[...] = l_s[...] * alpha + jnp.sum(e, axis=0, keepdims=True)
        ev = _bf16_dot(xc, e, ((0,), (0,)))
        accx_s[...] = accx_s[...] * alpha + ev

        @pl.when(j == nch - 1)
        def _fin_nx():
            nx_out[...] = (accx_s[...] / l_s[...]).T


def kernel(p, x, o, S):
    n = p.shape[0]
    n_seg = o.shape[0]
    seg_len = n // n_seg
    n_s = S.shape[0]
    c = x.shape[1]
    ch = _choose_chunk(seg_len)
    nch = seg_len // ch

    body = functools.partial(_attn_body, nch=nch, gamma=_GAMMA)
    n_p, n_x = pl.pallas_call(
        body,
        grid=(n_seg, 3, nch),
        in_specs=[
            pl.BlockSpec((ch, 3), lambda i, s, j: (i * nch + j, 0)),
            pl.BlockSpec((ch, c), lambda i, s, j: (i * nch + j, 0)),
            pl.BlockSpec((n_s, c), lambda i, s, j: (0, 0)),
        ],
        out_specs=[
            pl.BlockSpec((n_s, 3), lambda i, s, j: (i, 0)),
            pl.BlockSpec((n_s, c), lambda i, s, j: (i, 0)),
        ],
        out_shape=[
            jax.ShapeDtypeStruct((n_seg * n_s, 3), jnp.float32),
            jax.ShapeDtypeStruct((n_seg * n_s, c), jnp.float32),
        ],
        scratch_shapes=[
            pltpu.VMEM((1, n_s), jnp.float32),
            pltpu.VMEM((1, n_s), jnp.float32),
            pltpu.VMEM((3, n_s), jnp.float32),
            pltpu.VMEM((c, n_s), jnp.float32),
            pltpu.VMEM((3, n_s), jnp.float32),
            pltpu.VMEM((1, n_s), jnp.float32),
        ],
    )(p, x.astype(jnp.bfloat16), S.astype(jnp.bfloat16))

    n_o = jnp.arange(1, n_seg + 1, dtype=jnp.int32) * jnp.int32(n_s)
    return (n_p, n_x, n_o)


# speedup vs baseline: 1.3745x; 1.1666x over previous
import functools

import jax
import jax.numpy as jnp
from jax.experimental import pallas as pl
from jax.experimental.pallas import tpu as pltpu

_GAMMA = 16.0


def _choose_chunk(seg_len: int, max_chunk: int = 10000) -> int:
    best = 0
    for ch in range(8, min(seg_len, max_chunk) + 1, 8):
        if seg_len % ch == 0:
            best = ch
    return best if best else seg_len


def _bf16_dot(a, b, contract):
    return jax.lax.dot_general(
        a.astype(jnp.bfloat16), b.astype(jnp.bfloat16),
        (contract, ((), ())),
        preferred_element_type=jnp.float32)


def _ones_sum(a):
    ones = jnp.ones((1, a.shape[0]), jnp.float32)
    return jax.lax.dot_general(ones, a, (((1,), (0,)), ((), ())),
                               precision=jax.lax.Precision.HIGHEST,
                               preferred_element_type=jnp.float32)


def _attn_body(p_ref, x_ref, s_ref, np_out, nx_out,
               m_s, l_s, accp_s, accx_s, np_s, pn2_s, *, nch, gamma):
    s = pl.program_id(1)
    j = pl.program_id(2)
    S = s_ref[...]
    xc = x_ref[...]
    pc = p_ref[0].T

    w_h = _bf16_dot(xc, S, ((1,), (1,)))

    @pl.when(s == 0)
    def _sweep_max_sum():
        @pl.when(j == 0)
        def _():
            m_s[...] = jnp.full_like(m_s[...], -jnp.inf)
            l_s[...] = jnp.zeros_like(l_s[...])
        m_s[...] = jnp.maximum(m_s[...], jnp.max(w_h, axis=0, keepdims=True))
        e = jnp.exp(w_h - 20.0)
        l_s[...] = l_s[...] + jnp.sum(e, axis=0, keepdims=True)

        @pl.when(j == nch - 1)
        def _fin_l():
            l_s[...] = l_s[...] * jnp.exp(20.0 - m_s[...])

    @pl.when(s == 1)
    def _sweep_np():
        @pl.when(j == 0)
        def _():
            accp_s[...] = jnp.zeros_like(accp_s[...])
        probs = jnp.exp(w_h - m_s[...]) / l_s[...]
        accp_s[...] = accp_s[...] + _bf16_dot(pc, probs, ((0,), (0,)))

        @pl.when(j == nch - 1)
        def _fin_np():
            np_t = accp_s[...]
            np_s[...] = np_t
            pn2_s[...] = -gamma * jnp.sum(np_t * np_t, axis=0, keepdims=True)
            np_out[...] = np_t.T

    @pl.when(s == 2)
    def _sweep_nx():
        @pl.when(j == 0)
        def _():
            m_s[...] = jnp.full_like(m_s[...], -jnp.inf)
            l_s[...] = jnp.zeros_like(l_s[...])
            accx_s[...] = jnp.zeros_like(accx_s[...])
        cross = _bf16_dot(pc, np_s[...], ((1,), (0,)))
        psq = pc * pc
        negg3 = jnp.full((1, 3), -gamma, jnp.float32)
        pc2g = jax.lax.dot_general(psq, negg3, (((1,), (1,)), ((), ())),
                                   precision=jax.lax.Precision.HIGHEST,
                                   preferred_element_type=jnp.float32)
        nd = ((2.0 * gamma) * cross + pn2_s[...]) + pc2g
        w = jnp.exp(nd) * w_h
        m_old = m_s[...]
        m_new = jnp.maximum(m_old, jnp.max(w, axis=0, keepdims=True))
        alpha = jnp.exp(m_old - m_new)
        e = jnp.exp(w - m_new)
        m_s[...] = m_new
        l_s[...] = l_s[...] * alpha + jnp.sum(e, axis=0, keepdims=True)
        ev = _bf16_dot(xc, e, ((0,), (0,)))
        accx_s[...] = accx_s[...] * alpha + ev

        @pl.when(j == nch - 1)
        def _fin_nx():
            nx_out[...] = (accx_s[...] / l_s[...]).T


def kernel(p, x, o, S):
    n = p.shape[0]
    n_seg = o.shape[0]
    seg_len = n // n_seg
    n_s = S.shape[0]
    c = x.shape[1]
    ch = _choose_chunk(seg_len)
    nch = seg_len // ch

    body = functools.partial(_attn_body, nch=nch, gamma=_GAMMA)
    n_p, n_x = pl.pallas_call(
        body,
        grid=(n_seg, 3, nch),
        in_specs=[
            pl.BlockSpec((1, 3, ch), lambda i, s, j: (i * nch + j, 0, 0)),
            pl.BlockSpec((ch, c), lambda i, s, j: (i * nch + j, 0)),
            pl.BlockSpec((n_s, c), lambda i, s, j: (0, 0)),
        ],
        out_specs=[
            pl.BlockSpec((n_s, 3), lambda i, s, j: (i, 0)),
            pl.BlockSpec((n_s, c), lambda i, s, j: (i, 0)),
        ],
        out_shape=[
            jax.ShapeDtypeStruct((n_seg * n_s, 3), jnp.float32),
            jax.ShapeDtypeStruct((n_seg * n_s, c), jnp.float32),
        ],
        scratch_shapes=[
            pltpu.VMEM((1, n_s), jnp.float32),
            pltpu.VMEM((1, n_s), jnp.float32),
            pltpu.VMEM((3, n_s), jnp.float32),
            pltpu.VMEM((c, n_s), jnp.float32),
            pltpu.VMEM((3, n_s), jnp.float32),
            pltpu.VMEM((1, n_s), jnp.float32),
        ],
    )(p.T.reshape(3, n // ch, ch).swapaxes(0, 1),
      x.astype(jnp.bfloat16), S.astype(jnp.bfloat16))

    n_o = jnp.arange(1, n_seg + 1, dtype=jnp.int32) * jnp.int32(n_s)
    return (n_p, n_x, n_o)
